# baseline (speedup 1.0000x reference)
import jax
import jax.numpy as jnp
from jax.experimental import pallas as pl
from jax.experimental.pallas import tpu as pltpu

_D = 768
_F = 2048
_E = 8
_R = 8
_N = 2048
_FT = 256
_NFF = _F // _FT


def _fused_kernel(x_ref, gw_ref, w1_ref, w3_ref, w2_ref, af_ref, bf_ref,
                  ex_ref, out_ref, logits_ref, xb_ref, p1_ref, p3_ref):
    j = pl.program_id(0)
    par = jax.lax.rem(j, 2)
    prev = jax.lax.rem(j + 1, 2)

    @pl.when(j == 0)
    def _cast_x():
        xb_ref[...] = x_ref[...].astype(jnp.bfloat16)

    @pl.when(j < _NFF)
    def _produce():
        xb = xb_ref[...]
        p1_ref[par] = jax.lax.dot_general(
            xb, w1_ref[...].astype(jnp.bfloat16), (((1,), (1,)), ((), ())),
            preferred_element_type=jnp.float32)
        p3_ref[par] = jax.lax.dot_general(
            xb, w3_ref[...].astype(jnp.bfloat16), (((1,), (1,)), ((), ())),
            preferred_element_type=jnp.float32)

    @pl.when(j > 0)
    def _consume():
        h1 = p1_ref[prev]
        h = (h1 * jax.nn.sigmoid(h1)) * p3_ref[prev]
        contrib = jax.lax.dot_general(
            h.astype(jnp.bfloat16), w2_ref[...].astype(jnp.bfloat16),
            (((1,), (1,)), ((), ())),
            preferred_element_type=jnp.float32)

        @pl.when(j == 1)
        def _init():
            out_ref[...] = contrib

        @pl.when(j > 1)
        def _acc():
            out_ref[...] += contrib

    @pl.when(j == _NFF)
    def _router_and_lora():
        x = x_ref[...]

        logits = jax.lax.dot_general(
            x, gw_ref[...], (((1,), (1,)), ((), ())),
            preferred_element_type=jnp.float32)
        logits_ref[...] = logits

        eidx = jax.lax.broadcasted_iota(jnp.int32, logits.shape, 1)
        m1 = jnp.max(logits, axis=-1, keepdims=True)
        i1 = jnp.min(jnp.where(logits == m1, eidx, _E), axis=-1,
                     keepdims=True)
        sel1 = eidx == i1
        masked = jnp.where(sel1, -jnp.inf, logits)
        m2 = jnp.max(masked, axis=-1, keepdims=True)
        i2 = jnp.min(jnp.where(masked == m2, eidx, _E), axis=-1,
                     keepdims=True)
        sel2 = eidx == i2
        v2 = jnp.exp(m2 - m1)
        inv = 1.0 / (1.0 + v2)
        wdense = jnp.where(sel1, inv, 0.0) + jnp.where(sel2, v2 * inv, 0.0)

        wexp = jax.lax.dot_general(
            wdense, ex_ref[...], (((1,), (0,)), ((), ())),
            preferred_element_type=jnp.float32)

        t = jax.lax.dot_general(
            x, af_ref[...], (((1,), (1,)), ((), ())),
            preferred_element_type=jnp.float32)
        out_ref[...] += jax.lax.dot_general(
            wexp * t, bf_ref[...], (((1,), (0,)), ((), ())),
            preferred_element_type=jnp.float32)


def kernel(data, gate_w, w1, w3, w2, lora_A, lora_B):
    a_flat = lora_A.reshape(_E * _R, _D)
    b_flat = lora_B.transpose(0, 2, 1).reshape(_E * _R, _D)
    expand = jnp.repeat(jnp.eye(_E, dtype=jnp.float32), _R, axis=1)

    grid = (_NFF + 1,)
    out, logits = pl.pallas_call(
        _fused_kernel,
        grid=grid,
        in_specs=[
            pl.BlockSpec((_N, _D), lambda j: (0, 0)),
            pl.BlockSpec((_E, _D), lambda j: (0, 0)),
            pl.BlockSpec((_FT, _D),
                         lambda j: (jnp.minimum(j, _NFF - 1), 0)),
            pl.BlockSpec((_FT, _D),
                         lambda j: (jnp.minimum(j, _NFF - 1), 0)),
            pl.BlockSpec((_D, _FT),
                         lambda j: (0, jnp.maximum(j - 1, 0))),
            pl.BlockSpec((_E * _R, _D), lambda j: (0, 0)),
            pl.BlockSpec((_E * _R, _D), lambda j: (0, 0)),
            pl.BlockSpec((_E, _E * _R), lambda j: (0, 0)),
        ],
        out_specs=[
            pl.BlockSpec((_N, _D), lambda j: (0, 0)),
            pl.BlockSpec((_N, _E), lambda j: (0, 0)),
        ],
        out_shape=[
            jax.ShapeDtypeStruct((_N, _D), jnp.float32),
            jax.ShapeDtypeStruct((_N, _E), jnp.float32),
        ],
        scratch_shapes=[
            pltpu.VMEM((_N, _D), jnp.bfloat16),
            pltpu.VMEM((2, _N, _FT), jnp.float32),
            pltpu.VMEM((2, _N, _FT), jnp.float32),
        ],
    )(data, gate_w, w1, w3, w2, a_flat, b_flat, expand)
    return out, logits


# speedup vs baseline: 1.2318x; 1.2318x over previous
import jax
import jax.numpy as jnp
from jax.experimental import pallas as pl
from jax.experimental.pallas import tpu as pltpu

_D = 768
_F = 2048
_E = 8
_R = 8
_N = 2048
_FT = 512


def _fused_kernel(x_ref, gw_ref, w1_ref, w3_ref, w2_ref, af_ref, bf_ref,
                  ex_ref, out_ref, logits_ref, xb_ref):
    j = pl.program_id(0)

    @pl.when(j == 0)
    def _prologue():
        x = x_ref[...]
        xb_ref[...] = x.astype(jnp.bfloat16)

        logits = jax.lax.dot_general(
            x, gw_ref[...], (((1,), (1,)), ((), ())),
            preferred_element_type=jnp.float32)
        logits_ref[...] = logits

        eidx = jax.lax.broadcasted_iota(jnp.int32, logits.shape, 1)
        m1 = jnp.max(logits, axis=-1, keepdims=True)
        i1 = jnp.min(jnp.where(logits == m1, eidx, _E), axis=-1,
                     keepdims=True)
        sel1 = eidx == i1
        masked = jnp.where(sel1, -jnp.inf, logits)
        m2 = jnp.max(masked, axis=-1, keepdims=True)
        i2 = jnp.min(jnp.where(masked == m2, eidx, _E), axis=-1,
                     keepdims=True)
        sel2 = eidx == i2
        v2 = jnp.exp(m2 - m1)
        inv = 1.0 / (1.0 + v2)
        wdense = jnp.where(sel1, inv, 0.0) + jnp.where(sel2, v2 * inv, 0.0)

        wexp = jax.lax.dot_general(
            wdense, ex_ref[...], (((1,), (0,)), ((), ())),
            preferred_element_type=jnp.float32)

        t = jax.lax.dot_general(
            x, af_ref[...], (((1,), (1,)), ((), ())),
            preferred_element_type=jnp.float32)
        out_ref[...] = jax.lax.dot_general(
            wexp * t, bf_ref[...], (((1,), (0,)), ((), ())),
            preferred_element_type=jnp.float32)

    xb = xb_ref[...]
    h1 = jax.lax.dot_general(
        xb, w1_ref[...].astype(jnp.bfloat16), (((1,), (1,)), ((), ())),
        preferred_element_type=jnp.float32)
    h3 = jax.lax.dot_general(
        xb, w3_ref[...].astype(jnp.bfloat16), (((1,), (1,)), ((), ())),
        preferred_element_type=jnp.float32)
    h = (h1 * jax.nn.sigmoid(h1)) * h3
    out_ref[...] += jax.lax.dot_general(
        h.astype(jnp.bfloat16), w2_ref[...].astype(jnp.bfloat16),
        (((1,), (1,)), ((), ())),
        preferred_element_type=jnp.float32)


def kernel(data, gate_w, w1, w3, w2, lora_A, lora_B):
    a_flat = lora_A.reshape(_E * _R, _D)
    b_flat = lora_B.transpose(0, 2, 1).reshape(_E * _R, _D)
    expand = jnp.repeat(jnp.eye(_E, dtype=jnp.float32), _R, axis=1)

    grid = (_F // _FT,)
    out, logits = pl.pallas_call(
        _fused_kernel,
        grid=grid,
        in_specs=[
            pl.BlockSpec((_N, _D), lambda j: (0, 0)),
            pl.BlockSpec((_E, _D), lambda j: (0, 0)),
            pl.BlockSpec((_FT, _D), lambda j: (j, 0)),
            pl.BlockSpec((_FT, _D), lambda j: (j, 0)),
            pl.BlockSpec((_D, _FT), lambda j: (0, j)),
            pl.BlockSpec((_E * _R, _D), lambda j: (0, 0)),
            pl.BlockSpec((_E * _R, _D), lambda j: (0, 0)),
            pl.BlockSpec((_E, _E * _R), lambda j: (0, 0)),
        ],
        out_specs=[
            pl.BlockSpec((_N, _D), lambda j: (0, 0)),
            pl.BlockSpec((_N, _E), lambda j: (0, 0)),
        ],
        out_shape=[
            jax.ShapeDtypeStruct((_N, _D), jnp.float32),
            jax.ShapeDtypeStruct((_N, _E), jnp.float32),
        ],
        scratch_shapes=[pltpu.VMEM((_N, _D), jnp.bfloat16)],
    )(data, gate_w, w1, w3, w2, a_flat, b_flat, expand)
    return out, logits
